# baseline (speedup 1.0000x reference)
import functools

import jax
import jax.numpy as jnp
from jax import lax
from jax.experimental import pallas as pl
from jax.experimental.pallas import tpu as pltpu
from jax.experimental.pallas import tpu_sc as plsc

N_NODES = 10000
FDIM = 128
NC = 2
NS = 16
CHUNK = 128
NBUF = 4
DEG_GRP = 8

E_UNIT = NC * NS * CHUNK * DEG_GRP
NACC = 10112
ROWS = NACC // NS
NSUB = 4
SUB = 2528
SUB_ACC = SUB + 32
ROWS_P = SUB_ACC // NS
RB = 1000
GRID = N_NODES // RB


def _sc_mesh():
    return plsc.VectorSubcoreMesh(core_axis_name="c", subcore_axis_name="s")


DEG_EB = 2000


def _tc_deg(dst2d, n_hi):
    egrid = dst2d.shape[0]

    def body(dst_ref, out_ref):
        def step(i, acc):
            d = dst_ref[pl.ds(i, 1), :]
            hi = d // 128
            lo = d % 128
            oh_hi = (lax.broadcasted_iota(jnp.int32, (n_hi, DEG_EB), 0)
                     == hi).astype(jnp.float32)
            oh_loT = (lax.broadcasted_iota(jnp.int32, (128, DEG_EB), 0)
                      == lo).astype(jnp.float32)
            return acc + lax.dot_general(
                oh_hi, oh_loT, (((1,), (1,)), ((), ())),
                preferred_element_type=jnp.float32)

        out_ref[...] = lax.fori_loop(
            0, egrid, step, jnp.zeros((n_hi, 128), jnp.float32))

    return pl.pallas_call(
        body,
        grid=(1,),
        in_specs=[
            pl.BlockSpec((egrid, DEG_EB), lambda i: (0, 0)),
        ],
        out_specs=pl.BlockSpec((n_hi, 128), lambda i: (0, 0)),
        out_shape=jax.ShapeDtypeStruct((n_hi, 128), jnp.float32),
    )(dst2d)


def _make_pass_kernel(e_pad):
    nch = e_pad // (NS * CHUNK)
    nrounds = nch // NBUF

    @functools.partial(
        pl.kernel,
        out_type=jax.ShapeDtypeStruct((NSUB, SUB_ACC, 2 * FDIM), jnp.float32),
        mesh=_sc_mesh(),
        scratch_types=[
            pltpu.VMEM((nch, CHUNK), jnp.int32),
            pltpu.VMEM((nch, CHUNK), jnp.int32),
            [pltpu.VMEM((CHUNK, FDIM), jnp.float32) for _ in range(NBUF)],
            pltpu.VMEM_SHARED((SUB_ACC, FDIM), jnp.float32),
            [pltpu.SemaphoreType.DMA for _ in range(NBUF)],
            [pltpu.SemaphoreType.DMA for _ in range(NBUF)],
        ],
    )
    def pass_kernel(gidx_hbm, didx_hbm, zeros_hbm, y_hbm, out,
                    gi_v, di_v, bufs, acc, gsems, ssems):
        c = lax.axis_index("c")
        s = lax.axis_index("s")
        for p in range(NSUB):
            pltpu.sync_copy(gidx_hbm.at[p, c, s], gi_v)
            pltpu.sync_copy(didx_hbm.at[p, s], di_v)
            pltpu.sync_copy(
                zeros_hbm.at[pl.ds(s * ROWS_P, ROWS_P)],
                acc.at[pl.ds(s * ROWS_P, ROWS_P)],
            )
            plsc.subcore_barrier()

            @pl.loop(0, nrounds)
            def _round(g):
                gdescs = []
                for b in range(NBUF):
                    j = g * NBUF + b

                    @pl.when(g > 0)
                    def _wait_scatter(b=b):
                        pltpu.make_async_copy(
                            y_hbm.at[pl.ds(0, CHUNK)], bufs[b], ssems[b]
                        ).wait()

                    gdescs.append(
                        pltpu.async_copy(y_hbm.at[gi_v.at[j]], bufs[b],
                                         gsems[b])
                    )
                for b in range(NBUF):
                    j = g * NBUF + b
                    gdescs[b].wait()
                    pltpu.async_copy(bufs[b], acc.at[di_v.at[j]], ssems[b],
                                     add=True)

            for b in range(NBUF):
                pltpu.make_async_copy(
                    y_hbm.at[pl.ds(0, CHUNK)], bufs[b], ssems[b]
                ).wait()
            plsc.subcore_barrier()
            pltpu.sync_copy(
                acc.at[pl.ds(s * ROWS_P, ROWS_P)],
                out.at[p, pl.ds(s * ROWS_P, ROWS_P), pl.ds(c * FDIM, FDIM)],
            )

    return pass_kernel


def _tc_prescale(deg_2d, feature, condition):
    def body(deg_ref, f_ref, c_ref, y0_ref, y1_ref, dis_ref, inv_ref):
        d = deg_ref[...] + 1.0
        dis = lax.rsqrt(d)
        inv = 1.0 / d
        dis_ref[...] = jnp.broadcast_to(dis, (RB, 16))
        inv_ref[...] = jnp.broadcast_to(inv, (RB, 16))
        y0_ref[...] = f_ref[...] * dis
        y1_ref[...] = c_ref[...] * dis

    return pl.pallas_call(
        body,
        grid=(GRID,),
        in_specs=[
            pl.BlockSpec((RB, 1), lambda i: (i, 0)),
            pl.BlockSpec((RB, FDIM), lambda i: (i, 0)),
            pl.BlockSpec((RB, FDIM), lambda i: (i, 0)),
        ],
        out_specs=[
            pl.BlockSpec((RB, FDIM), lambda i: (i, 0)),
            pl.BlockSpec((RB, FDIM), lambda i: (i, 0)),
            pl.BlockSpec((RB, 16), lambda i: (i, 0)),
            pl.BlockSpec((RB, 16), lambda i: (i, 0)),
        ],
        out_shape=[
            jax.ShapeDtypeStruct((N_NODES, FDIM), jnp.float32),
            jax.ShapeDtypeStruct((N_NODES, FDIM), jnp.float32),
            jax.ShapeDtypeStruct((N_NODES, 16), jnp.float32),
            jax.ShapeDtypeStruct((N_NODES, 16), jnp.float32),
        ],
    )(deg_2d, feature, condition)


def _tc_hidden(S1, dis16, inv16, feature, condition, W_f2h, b_f2h, W_c2h, b_c2h):
    def body(s1_ref, dis_ref, inv_ref, f_ref, c_ref, wf_ref, bf_ref,
             wc_ref, bc_ref, h_ref, y20_ref, y21_ref):
        dcol = dis_ref[:, 0:1]
        icol = inv_ref[:, 0:1]
        aggf = dcol * s1_ref[:, 0:FDIM] + icol * f_ref[...]
        aggc = dcol * s1_ref[:, FDIM:2 * FDIM] + icol * c_ref[...]
        f2h = jnp.tanh(
            jnp.dot(aggf, wf_ref[...], preferred_element_type=jnp.float32)
            + bf_ref[...]
        )
        c2h = jnp.tanh(
            jnp.dot(aggc, wc_ref[...], preferred_element_type=jnp.float32)
            + bc_ref[...]
        )
        h_ref[:, 0:FDIM] = f2h
        h_ref[:, FDIM:2 * FDIM] = c2h
        y20_ref[...] = dcol * f2h
        y21_ref[...] = dcol * c2h

    return pl.pallas_call(
        body,
        grid=(GRID,),
        in_specs=[
            pl.BlockSpec((RB, 2 * FDIM), lambda i: (i, 0)),
            pl.BlockSpec((RB, 16), lambda i: (i, 0)),
            pl.BlockSpec((RB, 16), lambda i: (i, 0)),
            pl.BlockSpec((RB, FDIM), lambda i: (i, 0)),
            pl.BlockSpec((RB, FDIM), lambda i: (i, 0)),
            pl.BlockSpec((FDIM, FDIM), lambda i: (0, 0)),
            pl.BlockSpec((1, FDIM), lambda i: (0, 0)),
            pl.BlockSpec((FDIM, FDIM), lambda i: (0, 0)),
            pl.BlockSpec((1, FDIM), lambda i: (0, 0)),
        ],
        out_specs=[
            pl.BlockSpec((RB, 2 * FDIM), lambda i: (i, 0)),
            pl.BlockSpec((RB, FDIM), lambda i: (i, 0)),
            pl.BlockSpec((RB, FDIM), lambda i: (i, 0)),
        ],
        out_shape=[
            jax.ShapeDtypeStruct((N_NODES, 2 * FDIM), jnp.float32),
            jax.ShapeDtypeStruct((N_NODES, FDIM), jnp.float32),
            jax.ShapeDtypeStruct((N_NODES, FDIM), jnp.float32),
        ],
    )(S1, dis16, inv16, feature, condition, W_f2h, b_f2h, W_c2h, b_c2h)


def _tc_final(S2, dis16, inv16, h, W_mean, b_mean, W_logvar, b_logvar, noise):
    def body(s2_ref, dis_ref, inv_ref, h_ref, wm_ref, bm_ref, wl_ref, bl_ref,
             n_ref, z_ref, mean_ref, logvar_ref):
        dcol = dis_ref[:, 0:1]
        icol = inv_ref[:, 0:1]
        agg2 = dcol * s2_ref[...] + icol * h_ref[...]
        mean = jnp.dot(agg2, wm_ref[...], preferred_element_type=jnp.float32) \
            + bm_ref[...]
        logvar = jnp.dot(agg2, wl_ref[...], preferred_element_type=jnp.float32) \
            + bl_ref[...]
        z_ref[...] = n_ref[...] * jnp.exp(0.5 * logvar) + mean
        mean_ref[...] = mean
        logvar_ref[...] = logvar

    return pl.pallas_call(
        body,
        grid=(GRID,),
        in_specs=[
            pl.BlockSpec((RB, 2 * FDIM), lambda i: (i, 0)),
            pl.BlockSpec((RB, 16), lambda i: (i, 0)),
            pl.BlockSpec((RB, 16), lambda i: (i, 0)),
            pl.BlockSpec((RB, 2 * FDIM), lambda i: (i, 0)),
            pl.BlockSpec((2 * FDIM, FDIM), lambda i: (0, 0)),
            pl.BlockSpec((1, FDIM), lambda i: (0, 0)),
            pl.BlockSpec((2 * FDIM, FDIM), lambda i: (0, 0)),
            pl.BlockSpec((1, FDIM), lambda i: (0, 0)),
            pl.BlockSpec((RB, FDIM), lambda i: (i, 0)),
        ],
        out_specs=[
            pl.BlockSpec((RB, FDIM), lambda i: (i, 0)),
            pl.BlockSpec((RB, FDIM), lambda i: (i, 0)),
            pl.BlockSpec((RB, FDIM), lambda i: (i, 0)),
        ],
        out_shape=[
            jax.ShapeDtypeStruct((N_NODES, FDIM), jnp.float32),
            jax.ShapeDtypeStruct((N_NODES, FDIM), jnp.float32),
            jax.ShapeDtypeStruct((N_NODES, FDIM), jnp.float32),
        ],
    )(S2, dis16, inv16, h, W_mean, b_mean, W_logvar, b_logvar, noise)


def kernel(feature, condition, edge_index, W_f2h, b_f2h, W_c2h, b_c2h,
           W_mean, b_mean, W_logvar, b_logvar, noise):
    n = feature.shape[0]
    e = edge_index.shape[1]
    assert n == N_NODES
    e_pad = -(-e // E_UNIT) * E_UNIT
    pad = e_pad - e
    nch_p = e_pad // (NS * CHUNK)
    nch_d = e_pad // (NC * NS * CHUNK)

    src = edge_index[0]
    dst = edge_index[1]
    srcp = jnp.concatenate([src, jnp.zeros((pad,), jnp.int32)])
    dstp = jnp.concatenate([dst, jnp.full((pad,), n, jnp.int32)])
    valid = jnp.concatenate([
        jnp.ones((e,), jnp.bool_), jnp.zeros((pad,), jnp.bool_)])
    gidx = jnp.stack([
        jnp.stack([
            jnp.where(valid & (dstp >= p * SUB) & (dstp < (p + 1) * SUB),
                      srcp + c * n, 2 * n)
            for c in range(NC)
        ])
        for p in range(NSUB)
    ]).reshape(NSUB, NC, NS, nch_p, CHUNK)
    didx_pass = jnp.stack([
        jnp.where(valid & (dstp >= p * SUB) & (dstp < (p + 1) * SUB),
                  dstp - p * SUB, SUB)
        for p in range(NSUB)
    ]).reshape(NSUB, NS, nch_p, CHUNK)
    zeros = jnp.zeros((SUB_ACC, FDIM), jnp.float32)
    zrows = jnp.zeros((NS, FDIM), jnp.float32)

    n_hi = -(-n // 1024) * 8
    dst2d = dst.reshape(e // DEG_EB, DEG_EB)
    deg_hist = _tc_deg(dst2d, n_hi)
    deg_col = deg_hist.reshape(-1)[:n].reshape(n, 1)
    y0, y1, dis16, inv16 = _tc_prescale(deg_col, feature, condition)

    pass_kernel = _make_pass_kernel(e_pad)
    y1_stacked = jnp.concatenate([y0, y1, zrows], axis=0)
    bf = b_f2h.reshape(1, FDIM)
    bc = b_c2h.reshape(1, FDIM)

    bound = 2 + 0 * dst[0]

    def loop_cond(st):
        return st[0] < bound

    def loop_body(st):
        i, y, _, h_prev = st
        S_ = pass_kernel(gidx, didx_pass, zeros, y)
        S = S_[:, :SUB, :].reshape(NSUB * SUB, 2 * FDIM)[:n]

        def do_hidden(_):
            h, y20, y21 = _tc_hidden(S, dis16, inv16, feature, condition,
                                     W_f2h, bf, W_c2h, bc)
            return jnp.concatenate([y20, y21, zrows], axis=0), h

        def do_skip(_):
            return y, h_prev

        y_next, h_next = lax.switch(jnp.minimum(i, 1), [do_hidden, do_skip], 0)
        return (i + 1, y_next, S, h_next)

    zero_nf = jnp.zeros((n, 2 * FDIM), jnp.float32)
    _, _, S2, h = lax.while_loop(
        loop_cond, loop_body,
        (jnp.int32(0), y1_stacked, zero_nf, zero_nf))

    z, mean, logvar = _tc_final(S2, dis16, inv16, h,
                                W_mean, b_mean.reshape(1, FDIM),
                                W_logvar, b_logvar.reshape(1, FDIM), noise)
    return (z, mean, logvar)


# speedup vs baseline: 1.0008x; 1.0008x over previous
import functools

import jax
import jax.numpy as jnp
from jax import lax
from jax.experimental import pallas as pl
from jax.experimental.pallas import tpu as pltpu
from jax.experimental.pallas import tpu_sc as plsc

N_NODES = 10000
FDIM = 128
NC = 2
NS = 16
CHUNK = 128
NBUF = 4
SPLIT = 16
DEG_GRP = 8

E_UNIT = NC * NS * CHUNK * DEG_GRP
NACC = 10112
ROWS = NACC // NS
NSUB = 4
SUB = 2528
SUB_ACC = SUB + 32
ROWS_P = SUB_ACC // NS
RB = 1000
GRID = N_NODES // RB


def _sc_mesh():
    return plsc.VectorSubcoreMesh(core_axis_name="c", subcore_axis_name="s")


DEG_EB = 2000


def _tc_deg(dst2d, n_hi):
    egrid = dst2d.shape[0]

    def body(dst_ref, out_ref):
        def step(i, acc):
            d = dst_ref[pl.ds(i, 1), :]
            hi = d // 128
            lo = d % 128
            oh_hi = (lax.broadcasted_iota(jnp.int32, (n_hi, DEG_EB), 0)
                     == hi).astype(jnp.float32)
            oh_loT = (lax.broadcasted_iota(jnp.int32, (128, DEG_EB), 0)
                      == lo).astype(jnp.float32)
            return acc + lax.dot_general(
                oh_hi, oh_loT, (((1,), (1,)), ((), ())),
                preferred_element_type=jnp.float32)

        out_ref[...] = lax.fori_loop(
            0, egrid, step, jnp.zeros((n_hi, 128), jnp.float32))

    return pl.pallas_call(
        body,
        grid=(1,),
        in_specs=[
            pl.BlockSpec((egrid, DEG_EB), lambda i: (0, 0)),
        ],
        out_specs=pl.BlockSpec((n_hi, 128), lambda i: (0, 0)),
        out_shape=jax.ShapeDtypeStruct((n_hi, 128), jnp.float32),
    )(dst2d)


def _make_pass_kernel(e_pad):
    nch = e_pad // (NS * CHUNK)
    nrounds = nch // NBUF

    nsp = CHUNK // SPLIT

    @functools.partial(
        pl.kernel,
        out_type=jax.ShapeDtypeStruct((NSUB, SUB_ACC, 2 * FDIM), jnp.float32),
        mesh=_sc_mesh(),
        scratch_types=[
            pltpu.VMEM((nch, CHUNK), jnp.int32),
            pltpu.VMEM((nch, CHUNK), jnp.int32),
            [pltpu.VMEM((CHUNK, FDIM), jnp.float32) for _ in range(NBUF)],
            pltpu.VMEM_SHARED((SUB_ACC, FDIM), jnp.float32),
            [pltpu.SemaphoreType.DMA for _ in range(NBUF)],
            [pltpu.SemaphoreType.DMA for _ in range(NBUF)],
        ],
    )
    def pass_kernel(gidx_hbm, didx_hbm, zeros_hbm, y_hbm, out,
                    gi_v, di_v, bufs, acc, gsems, ssems):
        c = lax.axis_index("c")
        s = lax.axis_index("s")
        for p in range(NSUB):
            pltpu.sync_copy(gidx_hbm.at[p, c, s], gi_v)
            pltpu.sync_copy(didx_hbm.at[p, s], di_v)
            pltpu.sync_copy(
                zeros_hbm.at[pl.ds(s * ROWS_P, ROWS_P)],
                acc.at[pl.ds(s * ROWS_P, ROWS_P)],
            )
            plsc.subcore_barrier()

            @pl.loop(0, nrounds)
            def _round(g):
                gdescs = []
                for b in range(NBUF):
                    j = g * NBUF + b

                    @pl.when(g > 0)
                    def _wait_scatter(b=b):
                        pltpu.make_async_copy(
                            y_hbm.at[pl.ds(0, CHUNK)], bufs[b], ssems[b]
                        ).wait()

                    for k in range(nsp):
                        gdescs.append(pltpu.async_copy(
                            y_hbm.at[gi_v.at[j, pl.ds(k * SPLIT, SPLIT)]],
                            bufs[b].at[pl.ds(k * SPLIT, SPLIT)],
                            gsems[b]))
                for b in range(NBUF):
                    j = g * NBUF + b
                    for k in range(nsp):
                        gdescs[b * nsp + k].wait()
                        pltpu.async_copy(
                            bufs[b].at[pl.ds(k * SPLIT, SPLIT)],
                            acc.at[di_v.at[j, pl.ds(k * SPLIT, SPLIT)]],
                            ssems[b], add=True)

            for b in range(NBUF):
                pltpu.make_async_copy(
                    y_hbm.at[pl.ds(0, CHUNK)], bufs[b], ssems[b]
                ).wait()
            plsc.subcore_barrier()
            pltpu.sync_copy(
                acc.at[pl.ds(s * ROWS_P, ROWS_P)],
                out.at[p, pl.ds(s * ROWS_P, ROWS_P), pl.ds(c * FDIM, FDIM)],
            )

    return pass_kernel


def _tc_prescale(deg_2d, feature, condition):
    def body(deg_ref, f_ref, c_ref, y0_ref, y1_ref, dis_ref, inv_ref):
        d = deg_ref[...] + 1.0
        dis = lax.rsqrt(d)
        inv = 1.0 / d
        dis_ref[...] = jnp.broadcast_to(dis, (RB, 16))
        inv_ref[...] = jnp.broadcast_to(inv, (RB, 16))
        y0_ref[...] = f_ref[...] * dis
        y1_ref[...] = c_ref[...] * dis

    return pl.pallas_call(
        body,
        grid=(GRID,),
        in_specs=[
            pl.BlockSpec((RB, 1), lambda i: (i, 0)),
            pl.BlockSpec((RB, FDIM), lambda i: (i, 0)),
            pl.BlockSpec((RB, FDIM), lambda i: (i, 0)),
        ],
        out_specs=[
            pl.BlockSpec((RB, FDIM), lambda i: (i, 0)),
            pl.BlockSpec((RB, FDIM), lambda i: (i, 0)),
            pl.BlockSpec((RB, 16), lambda i: (i, 0)),
            pl.BlockSpec((RB, 16), lambda i: (i, 0)),
        ],
        out_shape=[
            jax.ShapeDtypeStruct((N_NODES, FDIM), jnp.float32),
            jax.ShapeDtypeStruct((N_NODES, FDIM), jnp.float32),
            jax.ShapeDtypeStruct((N_NODES, 16), jnp.float32),
            jax.ShapeDtypeStruct((N_NODES, 16), jnp.float32),
        ],
    )(deg_2d, feature, condition)


def _tc_hidden(S1, dis16, inv16, feature, condition, W_f2h, b_f2h, W_c2h, b_c2h):
    def body(s1_ref, dis_ref, inv_ref, f_ref, c_ref, wf_ref, bf_ref,
             wc_ref, bc_ref, h_ref, y20_ref, y21_ref):
        dcol = dis_ref[:, 0:1]
        icol = inv_ref[:, 0:1]
        aggf = dcol * s1_ref[:, 0:FDIM] + icol * f_ref[...]
        aggc = dcol * s1_ref[:, FDIM:2 * FDIM] + icol * c_ref[...]
        f2h = jnp.tanh(
            jnp.dot(aggf, wf_ref[...], preferred_element_type=jnp.float32)
            + bf_ref[...]
        )
        c2h = jnp.tanh(
            jnp.dot(aggc, wc_ref[...], preferred_element_type=jnp.float32)
            + bc_ref[...]
        )
        h_ref[:, 0:FDIM] = f2h
        h_ref[:, FDIM:2 * FDIM] = c2h
        y20_ref[...] = dcol * f2h
        y21_ref[...] = dcol * c2h

    return pl.pallas_call(
        body,
        grid=(GRID,),
        in_specs=[
            pl.BlockSpec((RB, 2 * FDIM), lambda i: (i, 0)),
            pl.BlockSpec((RB, 16), lambda i: (i, 0)),
            pl.BlockSpec((RB, 16), lambda i: (i, 0)),
            pl.BlockSpec((RB, FDIM), lambda i: (i, 0)),
            pl.BlockSpec((RB, FDIM), lambda i: (i, 0)),
            pl.BlockSpec((FDIM, FDIM), lambda i: (0, 0)),
            pl.BlockSpec((1, FDIM), lambda i: (0, 0)),
            pl.BlockSpec((FDIM, FDIM), lambda i: (0, 0)),
            pl.BlockSpec((1, FDIM), lambda i: (0, 0)),
        ],
        out_specs=[
            pl.BlockSpec((RB, 2 * FDIM), lambda i: (i, 0)),
            pl.BlockSpec((RB, FDIM), lambda i: (i, 0)),
            pl.BlockSpec((RB, FDIM), lambda i: (i, 0)),
        ],
        out_shape=[
            jax.ShapeDtypeStruct((N_NODES, 2 * FDIM), jnp.float32),
            jax.ShapeDtypeStruct((N_NODES, FDIM), jnp.float32),
            jax.ShapeDtypeStruct((N_NODES, FDIM), jnp.float32),
        ],
    )(S1, dis16, inv16, feature, condition, W_f2h, b_f2h, W_c2h, b_c2h)


def _tc_final(S2, dis16, inv16, h, W_mean, b_mean, W_logvar, b_logvar, noise):
    def body(s2_ref, dis_ref, inv_ref, h_ref, wm_ref, bm_ref, wl_ref, bl_ref,
             n_ref, z_ref, mean_ref, logvar_ref):
        dcol = dis_ref[:, 0:1]
        icol = inv_ref[:, 0:1]
        agg2 = dcol * s2_ref[...] + icol * h_ref[...]
        mean = jnp.dot(agg2, wm_ref[...], preferred_element_type=jnp.float32) \
            + bm_ref[...]
        logvar = jnp.dot(agg2, wl_ref[...], preferred_element_type=jnp.float32) \
            + bl_ref[...]
        z_ref[...] = n_ref[...] * jnp.exp(0.5 * logvar) + mean
        mean_ref[...] = mean
        logvar_ref[...] = logvar

    return pl.pallas_call(
        body,
        grid=(GRID,),
        in_specs=[
            pl.BlockSpec((RB, 2 * FDIM), lambda i: (i, 0)),
            pl.BlockSpec((RB, 16), lambda i: (i, 0)),
            pl.BlockSpec((RB, 16), lambda i: (i, 0)),
            pl.BlockSpec((RB, 2 * FDIM), lambda i: (i, 0)),
            pl.BlockSpec((2 * FDIM, FDIM), lambda i: (0, 0)),
            pl.BlockSpec((1, FDIM), lambda i: (0, 0)),
            pl.BlockSpec((2 * FDIM, FDIM), lambda i: (0, 0)),
            pl.BlockSpec((1, FDIM), lambda i: (0, 0)),
            pl.BlockSpec((RB, FDIM), lambda i: (i, 0)),
        ],
        out_specs=[
            pl.BlockSpec((RB, FDIM), lambda i: (i, 0)),
            pl.BlockSpec((RB, FDIM), lambda i: (i, 0)),
            pl.BlockSpec((RB, FDIM), lambda i: (i, 0)),
        ],
        out_shape=[
            jax.ShapeDtypeStruct((N_NODES, FDIM), jnp.float32),
            jax.ShapeDtypeStruct((N_NODES, FDIM), jnp.float32),
            jax.ShapeDtypeStruct((N_NODES, FDIM), jnp.float32),
        ],
    )(S2, dis16, inv16, h, W_mean, b_mean, W_logvar, b_logvar, noise)


def kernel(feature, condition, edge_index, W_f2h, b_f2h, W_c2h, b_c2h,
           W_mean, b_mean, W_logvar, b_logvar, noise):
    n = feature.shape[0]
    e = edge_index.shape[1]
    assert n == N_NODES
    e_pad = -(-e // E_UNIT) * E_UNIT
    pad = e_pad - e
    nch_p = e_pad // (NS * CHUNK)
    nch_d = e_pad // (NC * NS * CHUNK)

    src = edge_index[0]
    dst = edge_index[1]
    srcp = jnp.concatenate([src, jnp.zeros((pad,), jnp.int32)])
    dstp = jnp.concatenate([dst, jnp.full((pad,), n, jnp.int32)])
    valid = jnp.concatenate([
        jnp.ones((e,), jnp.bool_), jnp.zeros((pad,), jnp.bool_)])
    gidx = jnp.stack([
        jnp.stack([
            jnp.where(valid & (dstp >= p * SUB) & (dstp < (p + 1) * SUB),
                      srcp + c * n, 2 * n)
            for c in range(NC)
        ])
        for p in range(NSUB)
    ]).reshape(NSUB, NC, NS, nch_p, CHUNK)
    didx_pass = jnp.stack([
        jnp.where(valid & (dstp >= p * SUB) & (dstp < (p + 1) * SUB),
                  dstp - p * SUB, SUB)
        for p in range(NSUB)
    ]).reshape(NSUB, NS, nch_p, CHUNK)
    zeros = jnp.zeros((SUB_ACC, FDIM), jnp.float32)
    zrows = jnp.zeros((NS, FDIM), jnp.float32)

    n_hi = -(-n // 1024) * 8
    dst2d = dst.reshape(e // DEG_EB, DEG_EB)
    deg_hist = _tc_deg(dst2d, n_hi)
    deg_col = deg_hist.reshape(-1)[:n].reshape(n, 1)
    y0, y1, dis16, inv16 = _tc_prescale(deg_col, feature, condition)

    pass_kernel = _make_pass_kernel(e_pad)
    y1_stacked = jnp.concatenate([y0, y1, zrows], axis=0)
    bf = b_f2h.reshape(1, FDIM)
    bc = b_c2h.reshape(1, FDIM)

    bound = 2 + 0 * dst[0]

    def loop_cond(st):
        return st[0] < bound

    def loop_body(st):
        i, y, _, h_prev = st
        S_ = pass_kernel(gidx, didx_pass, zeros, y)
        S = S_[:, :SUB, :].reshape(NSUB * SUB, 2 * FDIM)[:n]

        def do_hidden(_):
            h, y20, y21 = _tc_hidden(S, dis16, inv16, feature, condition,
                                     W_f2h, bf, W_c2h, bc)
            return jnp.concatenate([y20, y21, zrows], axis=0), h

        def do_skip(_):
            return y, h_prev

        y_next, h_next = lax.switch(jnp.minimum(i, 1), [do_hidden, do_skip], 0)
        return (i + 1, y_next, S, h_next)

    zero_nf = jnp.zeros((n, 2 * FDIM), jnp.float32)
    _, _, S2, h = lax.while_loop(
        loop_cond, loop_body,
        (jnp.int32(0), y1_stacked, zero_nf, zero_nf))

    z, mean, logvar = _tc_final(S2, dis16, inv16, h,
                                W_mean, b_mean.reshape(1, FDIM),
                                W_logvar, b_logvar.reshape(1, FDIM), noise)
    return (z, mean, logvar)


# speedup vs baseline: 86.7428x; 86.6768x over previous
import functools

import jax
import jax.numpy as jnp
from jax import lax
from jax.experimental import pallas as pl
from jax.experimental.pallas import tpu as pltpu
from jax.experimental.pallas import tpu_sc as plsc

N_NODES = 10000
FDIM = 128
NC = 2
NS = 16
CHUNK = 128
NBUF = 4
SPLIT = 16
DEG_GRP = 8

E_UNIT = NC * NS * CHUNK * DEG_GRP
NACC = 10112
ROWS = NACC // NS
NSUB = 4
SUB = 2528
SUB_ACC = SUB + 32
ROWS_P = SUB_ACC // NS
ZPAD = 2048
RB = 1000
GRID = N_NODES // RB


def _sc_mesh():
    return plsc.VectorSubcoreMesh(core_axis_name="c", subcore_axis_name="s")


DEG_EB = 2000


def _tc_deg(dst2d, n_hi):
    egrid = dst2d.shape[0]

    def body(dst_ref, out_ref):
        def step(i, acc):
            d = dst_ref[pl.ds(i, 1), :]
            hi = d // 128
            lo = d % 128
            oh_hi = (lax.broadcasted_iota(jnp.int32, (n_hi, DEG_EB), 0)
                     == hi).astype(jnp.float32)
            oh_loT = (lax.broadcasted_iota(jnp.int32, (128, DEG_EB), 0)
                      == lo).astype(jnp.float32)
            return acc + lax.dot_general(
                oh_hi, oh_loT, (((1,), (1,)), ((), ())),
                preferred_element_type=jnp.float32)

        out_ref[...] = lax.fori_loop(
            0, egrid, step, jnp.zeros((n_hi, 128), jnp.float32))

    return pl.pallas_call(
        body,
        grid=(1,),
        in_specs=[
            pl.BlockSpec((egrid, DEG_EB), lambda i: (0, 0)),
        ],
        out_specs=pl.BlockSpec((n_hi, 128), lambda i: (0, 0)),
        out_shape=jax.ShapeDtypeStruct((n_hi, 128), jnp.float32),
    )(dst2d)


def _make_pass_kernel(e_pad):
    nch = e_pad // (NS * CHUNK)
    nrounds = nch // NBUF

    nsp = CHUNK // SPLIT

    @functools.partial(
        pl.kernel,
        out_type=jax.ShapeDtypeStruct((NSUB, SUB_ACC, 2 * FDIM), jnp.float32),
        mesh=_sc_mesh(),
        scratch_types=[
            pltpu.VMEM((nch, CHUNK), jnp.int32),
            pltpu.VMEM((nch, CHUNK), jnp.int32),
            [pltpu.VMEM((CHUNK, FDIM), jnp.float32) for _ in range(NBUF)],
            pltpu.VMEM_SHARED((SUB_ACC, FDIM), jnp.float32),
            [pltpu.SemaphoreType.DMA for _ in range(NBUF)],
            [pltpu.SemaphoreType.DMA for _ in range(NBUF)],
        ],
    )
    def pass_kernel(gidx_hbm, didx_hbm, zeros_hbm, y_hbm, out,
                    gi_v, di_v, bufs, acc, gsems, ssems):
        c = lax.axis_index("c")
        s = lax.axis_index("s")
        for p in range(NSUB):
            pltpu.sync_copy(gidx_hbm.at[p, c, s], gi_v)
            pltpu.sync_copy(didx_hbm.at[p, s], di_v)
            pltpu.sync_copy(
                zeros_hbm.at[pl.ds(s * ROWS_P, ROWS_P)],
                acc.at[pl.ds(s * ROWS_P, ROWS_P)],
            )
            plsc.subcore_barrier()

            @pl.loop(0, nrounds)
            def _round(g):
                gdescs = []
                for b in range(NBUF):
                    j = g * NBUF + b

                    @pl.when(g > 0)
                    def _wait_scatter(b=b):
                        pltpu.make_async_copy(
                            y_hbm.at[pl.ds(0, CHUNK)], bufs[b], ssems[b]
                        ).wait()

                    for k in range(nsp):
                        gdescs.append(pltpu.async_copy(
                            y_hbm.at[gi_v.at[j, pl.ds(k * SPLIT, SPLIT)]],
                            bufs[b].at[pl.ds(k * SPLIT, SPLIT)],
                            gsems[b]))
                for b in range(NBUF):
                    j = g * NBUF + b
                    for k in range(nsp):
                        gdescs[b * nsp + k].wait()
                        pltpu.async_copy(
                            bufs[b].at[pl.ds(k * SPLIT, SPLIT)],
                            acc.at[di_v.at[j, pl.ds(k * SPLIT, SPLIT)]],
                            ssems[b], add=True)

            for b in range(NBUF):
                pltpu.make_async_copy(
                    y_hbm.at[pl.ds(0, CHUNK)], bufs[b], ssems[b]
                ).wait()
            plsc.subcore_barrier()
            pltpu.sync_copy(
                acc.at[pl.ds(s * ROWS_P, ROWS_P)],
                out.at[p, pl.ds(s * ROWS_P, ROWS_P), pl.ds(c * FDIM, FDIM)],
            )

    return pass_kernel


def _tc_prescale(deg_2d, feature, condition):
    def body(deg_ref, f_ref, c_ref, y0_ref, y1_ref, dis_ref, inv_ref):
        d = deg_ref[...] + 1.0
        dis = lax.rsqrt(d)
        inv = 1.0 / d
        dis_ref[...] = jnp.broadcast_to(dis, (RB, 16))
        inv_ref[...] = jnp.broadcast_to(inv, (RB, 16))
        y0_ref[...] = f_ref[...] * dis
        y1_ref[...] = c_ref[...] * dis

    return pl.pallas_call(
        body,
        grid=(GRID,),
        in_specs=[
            pl.BlockSpec((RB, 1), lambda i: (i, 0)),
            pl.BlockSpec((RB, FDIM), lambda i: (i, 0)),
            pl.BlockSpec((RB, FDIM), lambda i: (i, 0)),
        ],
        out_specs=[
            pl.BlockSpec((RB, FDIM), lambda i: (i, 0)),
            pl.BlockSpec((RB, FDIM), lambda i: (i, 0)),
            pl.BlockSpec((RB, 16), lambda i: (i, 0)),
            pl.BlockSpec((RB, 16), lambda i: (i, 0)),
        ],
        out_shape=[
            jax.ShapeDtypeStruct((N_NODES, FDIM), jnp.float32),
            jax.ShapeDtypeStruct((N_NODES, FDIM), jnp.float32),
            jax.ShapeDtypeStruct((N_NODES, 16), jnp.float32),
            jax.ShapeDtypeStruct((N_NODES, 16), jnp.float32),
        ],
    )(deg_2d, feature, condition)


def _tc_hidden(S1, dis16, inv16, feature, condition, W_f2h, b_f2h, W_c2h, b_c2h):
    def body(s1_ref, dis_ref, inv_ref, f_ref, c_ref, wf_ref, bf_ref,
             wc_ref, bc_ref, h_ref, y20_ref, y21_ref):
        dcol = dis_ref[:, 0:1]
        icol = inv_ref[:, 0:1]
        aggf = dcol * s1_ref[:, 0:FDIM] + icol * f_ref[...]
        aggc = dcol * s1_ref[:, FDIM:2 * FDIM] + icol * c_ref[...]
        f2h = jnp.tanh(
            jnp.dot(aggf, wf_ref[...], preferred_element_type=jnp.float32)
            + bf_ref[...]
        )
        c2h = jnp.tanh(
            jnp.dot(aggc, wc_ref[...], preferred_element_type=jnp.float32)
            + bc_ref[...]
        )
        h_ref[:, 0:FDIM] = f2h
        h_ref[:, FDIM:2 * FDIM] = c2h
        y20_ref[...] = dcol * f2h
        y21_ref[...] = dcol * c2h

    return pl.pallas_call(
        body,
        grid=(GRID,),
        in_specs=[
            pl.BlockSpec((RB, 2 * FDIM), lambda i: (i, 0)),
            pl.BlockSpec((RB, 16), lambda i: (i, 0)),
            pl.BlockSpec((RB, 16), lambda i: (i, 0)),
            pl.BlockSpec((RB, FDIM), lambda i: (i, 0)),
            pl.BlockSpec((RB, FDIM), lambda i: (i, 0)),
            pl.BlockSpec((FDIM, FDIM), lambda i: (0, 0)),
            pl.BlockSpec((1, FDIM), lambda i: (0, 0)),
            pl.BlockSpec((FDIM, FDIM), lambda i: (0, 0)),
            pl.BlockSpec((1, FDIM), lambda i: (0, 0)),
        ],
        out_specs=[
            pl.BlockSpec((RB, 2 * FDIM), lambda i: (i, 0)),
            pl.BlockSpec((RB, FDIM), lambda i: (i, 0)),
            pl.BlockSpec((RB, FDIM), lambda i: (i, 0)),
        ],
        out_shape=[
            jax.ShapeDtypeStruct((N_NODES, 2 * FDIM), jnp.float32),
            jax.ShapeDtypeStruct((N_NODES, FDIM), jnp.float32),
            jax.ShapeDtypeStruct((N_NODES, FDIM), jnp.float32),
        ],
    )(S1, dis16, inv16, feature, condition, W_f2h, b_f2h, W_c2h, b_c2h)


def _tc_final(S2, dis16, inv16, h, W_mean, b_mean, W_logvar, b_logvar, noise):
    def body(s2_ref, dis_ref, inv_ref, h_ref, wm_ref, bm_ref, wl_ref, bl_ref,
             n_ref, z_ref, mean_ref, logvar_ref):
        dcol = dis_ref[:, 0:1]
        icol = inv_ref[:, 0:1]
        agg2 = dcol * s2_ref[...] + icol * h_ref[...]
        mean = jnp.dot(agg2, wm_ref[...], preferred_element_type=jnp.float32) \
            + bm_ref[...]
        logvar = jnp.dot(agg2, wl_ref[...], preferred_element_type=jnp.float32) \
            + bl_ref[...]
        z_ref[...] = n_ref[...] * jnp.exp(0.5 * logvar) + mean
        mean_ref[...] = mean
        logvar_ref[...] = logvar

    return pl.pallas_call(
        body,
        grid=(GRID,),
        in_specs=[
            pl.BlockSpec((RB, 2 * FDIM), lambda i: (i, 0)),
            pl.BlockSpec((RB, 16), lambda i: (i, 0)),
            pl.BlockSpec((RB, 16), lambda i: (i, 0)),
            pl.BlockSpec((RB, 2 * FDIM), lambda i: (i, 0)),
            pl.BlockSpec((2 * FDIM, FDIM), lambda i: (0, 0)),
            pl.BlockSpec((1, FDIM), lambda i: (0, 0)),
            pl.BlockSpec((2 * FDIM, FDIM), lambda i: (0, 0)),
            pl.BlockSpec((1, FDIM), lambda i: (0, 0)),
            pl.BlockSpec((RB, FDIM), lambda i: (i, 0)),
        ],
        out_specs=[
            pl.BlockSpec((RB, FDIM), lambda i: (i, 0)),
            pl.BlockSpec((RB, FDIM), lambda i: (i, 0)),
            pl.BlockSpec((RB, FDIM), lambda i: (i, 0)),
        ],
        out_shape=[
            jax.ShapeDtypeStruct((N_NODES, FDIM), jnp.float32),
            jax.ShapeDtypeStruct((N_NODES, FDIM), jnp.float32),
            jax.ShapeDtypeStruct((N_NODES, FDIM), jnp.float32),
        ],
    )(S2, dis16, inv16, h, W_mean, b_mean, W_logvar, b_logvar, noise)


def kernel(feature, condition, edge_index, W_f2h, b_f2h, W_c2h, b_c2h,
           W_mean, b_mean, W_logvar, b_logvar, noise):
    n = feature.shape[0]
    e = edge_index.shape[1]
    assert n == N_NODES
    e_pad = -(-e // E_UNIT) * E_UNIT
    pad = e_pad - e
    nch_p = e_pad // (NS * CHUNK)
    nch_d = e_pad // (NC * NS * CHUNK)

    src = edge_index[0]
    dst = edge_index[1]
    srcp = jnp.concatenate([src, jnp.zeros((pad,), jnp.int32)])
    dstp = jnp.concatenate([dst, jnp.full((pad,), n, jnp.int32)])
    valid = jnp.concatenate([
        jnp.ones((e,), jnp.bool_), jnp.zeros((pad,), jnp.bool_)])
    ar = jnp.arange(e_pad, dtype=jnp.int32)
    zspread = 2 * n + (ar % ZPAD)
    tspread = ar % SUB_ACC
    gidx = jnp.stack([
        jnp.stack([
            jnp.where(valid & (dstp >= p * SUB) & (dstp < (p + 1) * SUB),
                      srcp + c * n, zspread)
            for c in range(NC)
        ])
        for p in range(NSUB)
    ]).reshape(NSUB, NC, NS, nch_p, CHUNK)
    didx_pass = jnp.stack([
        jnp.where(valid & (dstp >= p * SUB) & (dstp < (p + 1) * SUB),
                  dstp - p * SUB, tspread)
        for p in range(NSUB)
    ]).reshape(NSUB, NS, nch_p, CHUNK)
    zeros = jnp.zeros((SUB_ACC, FDIM), jnp.float32)
    zrows = jnp.zeros((ZPAD, FDIM), jnp.float32)

    n_hi = -(-n // 1024) * 8
    dst2d = dst.reshape(e // DEG_EB, DEG_EB)
    deg_hist = _tc_deg(dst2d, n_hi)
    deg_col = deg_hist.reshape(-1)[:n].reshape(n, 1)
    y0, y1, dis16, inv16 = _tc_prescale(deg_col, feature, condition)

    pass_kernel = _make_pass_kernel(e_pad)
    y1_stacked = jnp.concatenate([y0, y1, zrows], axis=0)
    bf = b_f2h.reshape(1, FDIM)
    bc = b_c2h.reshape(1, FDIM)

    bound = 2 + 0 * dst[0]

    def loop_cond(st):
        return st[0] < bound

    def loop_body(st):
        i, y, _, h_prev = st
        S_ = pass_kernel(gidx, didx_pass, zeros, y)
        S = S_[:, :SUB, :].reshape(NSUB * SUB, 2 * FDIM)[:n]

        def do_hidden(_):
            h, y20, y21 = _tc_hidden(S, dis16, inv16, feature, condition,
                                     W_f2h, bf, W_c2h, bc)
            return jnp.concatenate([y20, y21, zrows], axis=0), h

        def do_skip(_):
            return y, h_prev

        y_next, h_next = lax.switch(jnp.minimum(i, 1), [do_hidden, do_skip], 0)
        return (i + 1, y_next, S, h_next)

    zero_nf = jnp.zeros((n, 2 * FDIM), jnp.float32)
    _, _, S2, h = lax.while_loop(
        loop_cond, loop_body,
        (jnp.int32(0), y1_stacked, zero_nf, zero_nf))

    z, mean, logvar = _tc_final(S2, dis16, inv16, h,
                                W_mean, b_mean.reshape(1, FDIM),
                                W_logvar, b_logvar.reshape(1, FDIM), noise)
    return (z, mean, logvar)


# speedup vs baseline: 86.7548x; 1.0001x over previous
import functools

import jax
import jax.numpy as jnp
from jax import lax
from jax.experimental import pallas as pl
from jax.experimental.pallas import tpu as pltpu
from jax.experimental.pallas import tpu_sc as plsc

N_NODES = 10000
FDIM = 128
NC = 2
NS = 16
CHUNK = 128
NBUF = 4
SPLIT = 16
E_UNIT = NC * NS * CHUNK * 8
NACC = 10112
ROWS = NACC // NS
NSUB = 4
SUB = 2528
SUB_ACC = SUB + 32
ROWS_P = SUB_ACC // NS
ZPAD = 2048
RB = 1000
GRID = N_NODES // RB


def _sc_mesh():
    return plsc.VectorSubcoreMesh(core_axis_name="c", subcore_axis_name="s")


DEG_EB = 2000


def _tc_deg(dst2d, n_hi):
    egrid = dst2d.shape[0]

    def body(dst_ref, out_ref):
        def step(i, acc):
            d = dst_ref[pl.ds(i, 1), :]
            hi = d // 128
            lo = d % 128
            oh_hi = (lax.broadcasted_iota(jnp.int32, (n_hi, DEG_EB), 0)
                     == hi).astype(jnp.float32)
            oh_loT = (lax.broadcasted_iota(jnp.int32, (128, DEG_EB), 0)
                      == lo).astype(jnp.float32)
            return acc + lax.dot_general(
                oh_hi, oh_loT, (((1,), (1,)), ((), ())),
                preferred_element_type=jnp.float32)

        out_ref[...] = lax.fori_loop(
            0, egrid, step, jnp.zeros((n_hi, 128), jnp.float32))

    return pl.pallas_call(
        body,
        grid=(1,),
        in_specs=[
            pl.BlockSpec((egrid, DEG_EB), lambda i: (0, 0)),
        ],
        out_specs=pl.BlockSpec((n_hi, 128), lambda i: (0, 0)),
        out_shape=jax.ShapeDtypeStruct((n_hi, 128), jnp.float32),
    )(dst2d)


def _make_pass_kernel(e_pad):
    nch = e_pad // (NS * CHUNK)
    nrounds = nch // NBUF

    nsp = CHUNK // SPLIT

    @functools.partial(
        pl.kernel,
        out_type=jax.ShapeDtypeStruct((NSUB, SUB_ACC, 2 * FDIM), jnp.float32),
        mesh=_sc_mesh(),
        scratch_types=[
            pltpu.VMEM((nch, CHUNK), jnp.int32),
            pltpu.VMEM((nch, CHUNK), jnp.int32),
            [pltpu.VMEM((CHUNK, FDIM), jnp.float32) for _ in range(NBUF)],
            pltpu.VMEM_SHARED((SUB_ACC, FDIM), jnp.float32),
            [pltpu.SemaphoreType.DMA for _ in range(NBUF)],
            [pltpu.SemaphoreType.DMA for _ in range(NBUF)],
        ],
    )
    def pass_kernel(gidx_hbm, didx_hbm, zeros_hbm, y_hbm, out,
                    gi_v, di_v, bufs, acc, gsems, ssems):
        c = lax.axis_index("c")
        s = lax.axis_index("s")
        for p in range(NSUB):
            pltpu.sync_copy(gidx_hbm.at[p, c, s], gi_v)
            pltpu.sync_copy(didx_hbm.at[p, s], di_v)
            pltpu.sync_copy(
                zeros_hbm.at[pl.ds(s * ROWS_P, ROWS_P)],
                acc.at[pl.ds(s * ROWS_P, ROWS_P)],
            )
            plsc.subcore_barrier()

            @pl.loop(0, nrounds)
            def _round(g):
                gdescs = []
                for b in range(NBUF):
                    j = g * NBUF + b

                    @pl.when(g > 0)
                    def _wait_scatter(b=b):
                        pltpu.make_async_copy(
                            y_hbm.at[pl.ds(0, CHUNK)], bufs[b], ssems[b]
                        ).wait()

                    for k in range(nsp):
                        gdescs.append(pltpu.async_copy(
                            y_hbm.at[gi_v.at[j, pl.ds(k * SPLIT, SPLIT)]],
                            bufs[b].at[pl.ds(k * SPLIT, SPLIT)],
                            gsems[b]))
                for b in range(NBUF):
                    j = g * NBUF + b
                    for k in range(nsp):
                        gdescs[b * nsp + k].wait()
                        pltpu.async_copy(
                            bufs[b].at[pl.ds(k * SPLIT, SPLIT)],
                            acc.at[di_v.at[j, pl.ds(k * SPLIT, SPLIT)]],
                            ssems[b], add=True)

            for b in range(NBUF):
                pltpu.make_async_copy(
                    y_hbm.at[pl.ds(0, CHUNK)], bufs[b], ssems[b]
                ).wait()
            plsc.subcore_barrier()
            pltpu.sync_copy(
                acc.at[pl.ds(s * ROWS_P, ROWS_P)],
                out.at[p, pl.ds(s * ROWS_P, ROWS_P), pl.ds(c * FDIM, FDIM)],
            )

    return pass_kernel


def _tc_prescale(deg_2d, feature, condition):
    def body(deg_ref, f_ref, c_ref, y0_ref, y1_ref, dis_ref, inv_ref):
        d = deg_ref[...] + 1.0
        dis = lax.rsqrt(d)
        inv = 1.0 / d
        dis_ref[...] = jnp.broadcast_to(dis, (RB, 16))
        inv_ref[...] = jnp.broadcast_to(inv, (RB, 16))
        y0_ref[...] = f_ref[...] * dis
        y1_ref[...] = c_ref[...] * dis

    return pl.pallas_call(
        body,
        grid=(GRID,),
        in_specs=[
            pl.BlockSpec((RB, 1), lambda i: (i, 0)),
            pl.BlockSpec((RB, FDIM), lambda i: (i, 0)),
            pl.BlockSpec((RB, FDIM), lambda i: (i, 0)),
        ],
        out_specs=[
            pl.BlockSpec((RB, FDIM), lambda i: (i, 0)),
            pl.BlockSpec((RB, FDIM), lambda i: (i, 0)),
            pl.BlockSpec((RB, 16), lambda i: (i, 0)),
            pl.BlockSpec((RB, 16), lambda i: (i, 0)),
        ],
        out_shape=[
            jax.ShapeDtypeStruct((N_NODES, FDIM), jnp.float32),
            jax.ShapeDtypeStruct((N_NODES, FDIM), jnp.float32),
            jax.ShapeDtypeStruct((N_NODES, 16), jnp.float32),
            jax.ShapeDtypeStruct((N_NODES, 16), jnp.float32),
        ],
    )(deg_2d, feature, condition)


def _tc_hidden(S1, dis16, inv16, feature, condition, W_f2h, b_f2h, W_c2h, b_c2h):
    def body(s1_ref, dis_ref, inv_ref, f_ref, c_ref, wf_ref, bf_ref,
             wc_ref, bc_ref, h_ref, y20_ref, y21_ref):
        dcol = dis_ref[:, 0:1]
        icol = inv_ref[:, 0:1]
        aggf = dcol * s1_ref[:, 0:FDIM] + icol * f_ref[...]
        aggc = dcol * s1_ref[:, FDIM:2 * FDIM] + icol * c_ref[...]
        f2h = jnp.tanh(
            jnp.dot(aggf, wf_ref[...], preferred_element_type=jnp.float32)
            + bf_ref[...]
        )
        c2h = jnp.tanh(
            jnp.dot(aggc, wc_ref[...], preferred_element_type=jnp.float32)
            + bc_ref[...]
        )
        h_ref[:, 0:FDIM] = f2h
        h_ref[:, FDIM:2 * FDIM] = c2h
        y20_ref[...] = dcol * f2h
        y21_ref[...] = dcol * c2h

    return pl.pallas_call(
        body,
        grid=(GRID,),
        in_specs=[
            pl.BlockSpec((RB, 2 * FDIM), lambda i: (i, 0)),
            pl.BlockSpec((RB, 16), lambda i: (i, 0)),
            pl.BlockSpec((RB, 16), lambda i: (i, 0)),
            pl.BlockSpec((RB, FDIM), lambda i: (i, 0)),
            pl.BlockSpec((RB, FDIM), lambda i: (i, 0)),
            pl.BlockSpec((FDIM, FDIM), lambda i: (0, 0)),
            pl.BlockSpec((1, FDIM), lambda i: (0, 0)),
            pl.BlockSpec((FDIM, FDIM), lambda i: (0, 0)),
            pl.BlockSpec((1, FDIM), lambda i: (0, 0)),
        ],
        out_specs=[
            pl.BlockSpec((RB, 2 * FDIM), lambda i: (i, 0)),
            pl.BlockSpec((RB, FDIM), lambda i: (i, 0)),
            pl.BlockSpec((RB, FDIM), lambda i: (i, 0)),
        ],
        out_shape=[
            jax.ShapeDtypeStruct((N_NODES, 2 * FDIM), jnp.float32),
            jax.ShapeDtypeStruct((N_NODES, FDIM), jnp.float32),
            jax.ShapeDtypeStruct((N_NODES, FDIM), jnp.float32),
        ],
    )(S1, dis16, inv16, feature, condition, W_f2h, b_f2h, W_c2h, b_c2h)


def _tc_final(S2, dis16, inv16, h, W_mean, b_mean, W_logvar, b_logvar, noise):
    def body(s2_ref, dis_ref, inv_ref, h_ref, wm_ref, bm_ref, wl_ref, bl_ref,
             n_ref, z_ref, mean_ref, logvar_ref):
        dcol = dis_ref[:, 0:1]
        icol = inv_ref[:, 0:1]
        agg2 = dcol * s2_ref[...] + icol * h_ref[...]
        mean = jnp.dot(agg2, wm_ref[...], preferred_element_type=jnp.float32) \
            + bm_ref[...]
        logvar = jnp.dot(agg2, wl_ref[...], preferred_element_type=jnp.float32) \
            + bl_ref[...]
        z_ref[...] = n_ref[...] * jnp.exp(0.5 * logvar) + mean
        mean_ref[...] = mean
        logvar_ref[...] = logvar

    return pl.pallas_call(
        body,
        grid=(GRID,),
        in_specs=[
            pl.BlockSpec((RB, 2 * FDIM), lambda i: (i, 0)),
            pl.BlockSpec((RB, 16), lambda i: (i, 0)),
            pl.BlockSpec((RB, 16), lambda i: (i, 0)),
            pl.BlockSpec((RB, 2 * FDIM), lambda i: (i, 0)),
            pl.BlockSpec((2 * FDIM, FDIM), lambda i: (0, 0)),
            pl.BlockSpec((1, FDIM), lambda i: (0, 0)),
            pl.BlockSpec((2 * FDIM, FDIM), lambda i: (0, 0)),
            pl.BlockSpec((1, FDIM), lambda i: (0, 0)),
            pl.BlockSpec((RB, FDIM), lambda i: (i, 0)),
        ],
        out_specs=[
            pl.BlockSpec((RB, FDIM), lambda i: (i, 0)),
            pl.BlockSpec((RB, FDIM), lambda i: (i, 0)),
            pl.BlockSpec((RB, FDIM), lambda i: (i, 0)),
        ],
        out_shape=[
            jax.ShapeDtypeStruct((N_NODES, FDIM), jnp.float32),
            jax.ShapeDtypeStruct((N_NODES, FDIM), jnp.float32),
            jax.ShapeDtypeStruct((N_NODES, FDIM), jnp.float32),
        ],
    )(S2, dis16, inv16, h, W_mean, b_mean, W_logvar, b_logvar, noise)


def kernel(feature, condition, edge_index, W_f2h, b_f2h, W_c2h, b_c2h,
           W_mean, b_mean, W_logvar, b_logvar, noise):
    n = feature.shape[0]
    e = edge_index.shape[1]
    assert n == N_NODES
    e_pad = -(-e // E_UNIT) * E_UNIT
    pad = e_pad - e
    nch_p = e_pad // (NS * CHUNK)

    src = edge_index[0]
    dst = edge_index[1]
    srcp = jnp.concatenate([src, jnp.zeros((pad,), jnp.int32)])
    dstp = jnp.concatenate([dst, jnp.full((pad,), n, jnp.int32)])
    valid = jnp.concatenate([
        jnp.ones((e,), jnp.bool_), jnp.zeros((pad,), jnp.bool_)])
    ar = jnp.arange(e_pad, dtype=jnp.int32)
    zspread = 2 * n + (ar % ZPAD)
    tspread = ar % SUB_ACC
    gidx = jnp.stack([
        jnp.stack([
            jnp.where(valid & (dstp >= p * SUB) & (dstp < (p + 1) * SUB),
                      srcp + c * n, zspread)
            for c in range(NC)
        ])
        for p in range(NSUB)
    ]).reshape(NSUB, NC, NS, nch_p, CHUNK)
    didx_pass = jnp.stack([
        jnp.where(valid & (dstp >= p * SUB) & (dstp < (p + 1) * SUB),
                  dstp - p * SUB, tspread)
        for p in range(NSUB)
    ]).reshape(NSUB, NS, nch_p, CHUNK)
    zeros = jnp.zeros((SUB_ACC, FDIM), jnp.float32)
    zrows = jnp.zeros((ZPAD, FDIM), jnp.float32)

    n_hi = -(-n // 1024) * 8
    dst2d = dst.reshape(e // DEG_EB, DEG_EB)
    deg_hist = _tc_deg(dst2d, n_hi)
    deg_col = deg_hist.reshape(-1)[:n].reshape(n, 1)
    y0, y1, dis16, inv16 = _tc_prescale(deg_col, feature, condition)

    pass_kernel = _make_pass_kernel(e_pad)
    y1_stacked = jnp.concatenate([y0, y1, zrows], axis=0)
    bf = b_f2h.reshape(1, FDIM)
    bc = b_c2h.reshape(1, FDIM)

    bound = 2 + 0 * dst[0]

    def loop_cond(st):
        return st[0] < bound

    def loop_body(st):
        i, y, _, h_prev = st
        S_ = pass_kernel(gidx, didx_pass, zeros, y)
        S = S_[:, :SUB, :].reshape(NSUB * SUB, 2 * FDIM)[:n]

        def do_hidden(_):
            h, y20, y21 = _tc_hidden(S, dis16, inv16, feature, condition,
                                     W_f2h, bf, W_c2h, bc)
            return jnp.concatenate([y20, y21, zrows], axis=0), h

        def do_skip(_):
            return y, h_prev

        y_next, h_next = lax.switch(jnp.minimum(i, 1), [do_hidden, do_skip], 0)
        return (i + 1, y_next, S, h_next)

    zero_nf = jnp.zeros((n, 2 * FDIM), jnp.float32)
    _, _, S2, h = lax.while_loop(
        loop_cond, loop_body,
        (jnp.int32(0), y1_stacked, zero_nf, zero_nf))

    z, mean, logvar = _tc_final(S2, dis16, inv16, h,
                                W_mean, b_mean.reshape(1, FDIM),
                                W_logvar, b_logvar.reshape(1, FDIM), noise)
    return (z, mean, logvar)
